# initial kernel scaffold (unmeasured)
import jax
import jax.numpy as jnp
from jax import lax
from jax.experimental import pallas as pl
from jax.experimental.pallas import tpu as pltpu


def kernel(
    x,
):
    def body(*refs):
        pass

    out_shape = jax.ShapeDtypeStruct(..., jnp.float32)
    return pl.pallas_call(body, out_shape=out_shape)(...)


# baseline (device time: 229838 ns/iter reference)
import jax
import jax.numpy as jnp
from jax import lax
from jax.experimental import pallas as pl
from jax.experimental.pallas import tpu as pltpu

M_SHARD = 8192
N = 1024
N_Z = 2


def kernel(x):
    x_bf = x.astype(jnp.bfloat16)

    def body(x_ref, out_ref, local_sem, send_sem, recv_sem):
        my_x = lax.axis_index("x")
        my_y = lax.axis_index("y")
        my_z = lax.axis_index("z")
        peer = (my_x, my_y, 1 - my_z)

        barrier_sem = pltpu.get_barrier_semaphore()
        pl.semaphore_signal(
            barrier_sem, inc=1, device_id=peer,
            device_id_type=pl.DeviceIdType.MESH,
        )
        pl.semaphore_wait(barrier_sem, 1)

        local = pltpu.make_async_copy(
            x_ref, out_ref.at[pl.ds(my_z * M_SHARD, M_SHARD), :], local_sem
        )
        local.start()

        rdma = pltpu.make_async_remote_copy(
            src_ref=x_ref,
            dst_ref=out_ref.at[pl.ds(my_z * M_SHARD, M_SHARD), :],
            send_sem=send_sem,
            recv_sem=recv_sem,
            device_id=peer,
            device_id_type=pl.DeviceIdType.MESH,
        )
        rdma.start()
        rdma.wait()
        local.wait()

    return pl.pallas_call(
        body,
        out_shape=jax.ShapeDtypeStruct((N_Z * M_SHARD, N), jnp.bfloat16),
        in_specs=[pl.BlockSpec(memory_space=pltpu.VMEM)],
        out_specs=pl.BlockSpec(memory_space=pltpu.VMEM),
        scratch_shapes=[
            pltpu.SemaphoreType.DMA,
            pltpu.SemaphoreType.DMA,
            pltpu.SemaphoreType.DMA,
        ],
        compiler_params=pltpu.CompilerParams(collective_id=0),
    )(x_bf)


# device time: 153857 ns/iter; 1.4938x vs baseline; 1.4938x over previous
import jax
import jax.numpy as jnp
from jax import lax
from jax.experimental import pallas as pl
from jax.experimental.pallas import tpu as pltpu

M_SHARD = 8192
HALF = M_SHARD // 2
N = 1024
N_Z = 2
N_CHUNKS = 8
CH = HALF // N_CHUNKS


def kernel(x):
    x_bf = x.astype(jnp.bfloat16)

    def body(x_ref, out_ref, local_sem, z_send_sems, z_recv_sems,
             x_send_sems, x_recv_sems):
        my_x = lax.axis_index("x")
        my_y = lax.axis_index("y")
        my_z = lax.axis_index("z")
        z_peer = (my_x, my_y, 1 - my_z)
        x_peer = (1 - my_x, my_y, my_z)

        barrier_sem = pltpu.get_barrier_semaphore()
        for peer in (z_peer, x_peer):
            pl.semaphore_signal(
                barrier_sem, inc=1, device_id=peer,
                device_id_type=pl.DeviceIdType.MESH,
            )
        pl.semaphore_wait(barrier_sem, 2)

        local = pltpu.make_async_copy(
            x_ref, out_ref.at[pl.ds(my_z * M_SHARD, M_SHARD), :], local_sem
        )
        local.start()

        def my_half_rows(i):
            return my_z * M_SHARD + my_x * HALF + i * CH

        def z_recv_rows(i):
            return (1 - my_z) * M_SHARD + my_x * HALF + i * CH

        def x_recv_rows(i):
            return (1 - my_z) * M_SHARD + (1 - my_x) * HALF + i * CH

        z_rdmas = []
        for i in range(N_CHUNKS):
            r = pltpu.make_async_remote_copy(
                src_ref=x_ref.at[pl.ds(my_x * HALF + i * CH, CH), :],
                dst_ref=out_ref.at[pl.ds(my_half_rows(i), CH), :],
                send_sem=z_send_sems.at[i],
                recv_sem=z_recv_sems.at[i],
                device_id=z_peer,
                device_id_type=pl.DeviceIdType.MESH,
            )
            r.start()
            z_rdmas.append(r)

        x_rdmas = []
        for j in range(N_CHUNKS):
            z_rdmas[j].wait_recv()
            r = pltpu.make_async_remote_copy(
                src_ref=out_ref.at[pl.ds(z_recv_rows(j), CH), :],
                dst_ref=out_ref.at[pl.ds(z_recv_rows(j), CH), :],
                send_sem=x_send_sems.at[j],
                recv_sem=x_recv_sems.at[j],
                device_id=x_peer,
                device_id_type=pl.DeviceIdType.MESH,
            )
            r.start()
            x_rdmas.append(r)

        for j in range(N_CHUNKS):
            recv = pltpu.make_async_remote_copy(
                src_ref=out_ref.at[pl.ds(x_recv_rows(j), CH), :],
                dst_ref=out_ref.at[pl.ds(x_recv_rows(j), CH), :],
                send_sem=x_send_sems.at[j],
                recv_sem=x_recv_sems.at[j],
                device_id=x_peer,
                device_id_type=pl.DeviceIdType.MESH,
            )
            recv.wait_recv()
        for j in range(N_CHUNKS):
            z_rdmas[j].wait_send()
            x_rdmas[j].wait_send()
        local.wait()

    return pl.pallas_call(
        body,
        out_shape=jax.ShapeDtypeStruct((N_Z * M_SHARD, N), jnp.bfloat16),
        in_specs=[pl.BlockSpec(memory_space=pltpu.VMEM)],
        out_specs=pl.BlockSpec(memory_space=pltpu.VMEM),
        scratch_shapes=[
            pltpu.SemaphoreType.DMA,
            pltpu.SemaphoreType.DMA((N_CHUNKS,)),
            pltpu.SemaphoreType.DMA((N_CHUNKS,)),
            pltpu.SemaphoreType.DMA((N_CHUNKS,)),
            pltpu.SemaphoreType.DMA((N_CHUNKS,)),
        ],
        compiler_params=pltpu.CompilerParams(collective_id=0),
    )(x_bf)


# device time: 136952 ns/iter; 1.6782x vs baseline; 1.1234x over previous
import jax
import jax.numpy as jnp
from jax import lax
from jax.experimental import pallas as pl
from jax.experimental.pallas import tpu as pltpu

M_SHARD = 8192
HALF = M_SHARD // 2
N = 1024
N_Z = 2
N_CHUNKS = 16
CH = HALF // N_CHUNKS


def kernel(x):
    x_bf = x.astype(jnp.bfloat16)

    def body(x_ref, out_ref, local_sem, z_send_sems, z_recv_sems,
             x_send_sems, x_recv_sems):
        my_x = lax.axis_index("x")
        my_y = lax.axis_index("y")
        my_z = lax.axis_index("z")
        z_peer = (my_x, my_y, 1 - my_z)
        x_peer = (1 - my_x, my_y, my_z)

        barrier_sem = pltpu.get_barrier_semaphore()
        for peer in (z_peer, x_peer):
            pl.semaphore_signal(
                barrier_sem, inc=1, device_id=peer,
                device_id_type=pl.DeviceIdType.MESH,
            )
        pl.semaphore_wait(barrier_sem, 2)

        local = pltpu.make_async_copy(
            x_ref, out_ref.at[pl.ds(my_z * M_SHARD, M_SHARD), :], local_sem
        )
        local.start()

        def my_half_rows(i):
            return my_z * M_SHARD + my_x * HALF + i * CH

        def z_recv_rows(i):
            return (1 - my_z) * M_SHARD + my_x * HALF + i * CH

        def x_recv_rows(i):
            return (1 - my_z) * M_SHARD + (1 - my_x) * HALF + i * CH

        z_rdmas = []
        for i in range(N_CHUNKS):
            r = pltpu.make_async_remote_copy(
                src_ref=x_ref.at[pl.ds(my_x * HALF + i * CH, CH), :],
                dst_ref=out_ref.at[pl.ds(my_half_rows(i), CH), :],
                send_sem=z_send_sems.at[i],
                recv_sem=z_recv_sems.at[i],
                device_id=z_peer,
                device_id_type=pl.DeviceIdType.MESH,
            )
            r.start()
            z_rdmas.append(r)

        x_rdmas = []
        for j in range(N_CHUNKS):
            z_rdmas[j].wait_recv()
            r = pltpu.make_async_remote_copy(
                src_ref=out_ref.at[pl.ds(z_recv_rows(j), CH), :],
                dst_ref=out_ref.at[pl.ds(z_recv_rows(j), CH), :],
                send_sem=x_send_sems.at[j],
                recv_sem=x_recv_sems.at[j],
                device_id=x_peer,
                device_id_type=pl.DeviceIdType.MESH,
            )
            r.start()
            x_rdmas.append(r)

        for j in range(N_CHUNKS):
            recv = pltpu.make_async_remote_copy(
                src_ref=out_ref.at[pl.ds(x_recv_rows(j), CH), :],
                dst_ref=out_ref.at[pl.ds(x_recv_rows(j), CH), :],
                send_sem=x_send_sems.at[j],
                recv_sem=x_recv_sems.at[j],
                device_id=x_peer,
                device_id_type=pl.DeviceIdType.MESH,
            )
            recv.wait_recv()
        for j in range(N_CHUNKS):
            z_rdmas[j].wait_send()
            x_rdmas[j].wait_send()
        local.wait()

    return pl.pallas_call(
        body,
        out_shape=jax.ShapeDtypeStruct((N_Z * M_SHARD, N), jnp.bfloat16),
        in_specs=[pl.BlockSpec(memory_space=pl.ANY)],
        out_specs=pl.BlockSpec(memory_space=pl.ANY),
        scratch_shapes=[
            pltpu.SemaphoreType.DMA,
            pltpu.SemaphoreType.DMA((N_CHUNKS,)),
            pltpu.SemaphoreType.DMA((N_CHUNKS,)),
            pltpu.SemaphoreType.DMA((N_CHUNKS,)),
            pltpu.SemaphoreType.DMA((N_CHUNKS,)),
        ],
        compiler_params=pltpu.CompilerParams(collective_id=0),
    )(x_bf)


# device time: 135210 ns/iter; 1.6999x vs baseline; 1.0129x over previous
import jax
import jax.numpy as jnp
from jax import lax
from jax.experimental import pallas as pl
from jax.experimental.pallas import tpu as pltpu

M_SHARD = 8192
HALF = M_SHARD // 2
N = 1024
N_Z = 2
N_CHUNKS = 32
CH = HALF // N_CHUNKS


def kernel(x):
    x_bf = x.astype(jnp.bfloat16)

    def body(x_ref, out_ref, local_sem, z_send_sems, z_recv_sems,
             x_send_sems, x_recv_sems):
        my_x = lax.axis_index("x")
        my_y = lax.axis_index("y")
        my_z = lax.axis_index("z")
        z_peer = (my_x, my_y, 1 - my_z)
        x_peer = (1 - my_x, my_y, my_z)

        barrier_sem = pltpu.get_barrier_semaphore()
        for peer in (z_peer, x_peer):
            pl.semaphore_signal(
                barrier_sem, inc=1, device_id=peer,
                device_id_type=pl.DeviceIdType.MESH,
            )
        pl.semaphore_wait(barrier_sem, 2)

        local = pltpu.make_async_copy(
            x_ref, out_ref.at[pl.ds(my_z * M_SHARD, M_SHARD), :], local_sem
        )
        local.start()

        def my_half_rows(i):
            return my_z * M_SHARD + my_x * HALF + i * CH

        def z_recv_rows(i):
            return (1 - my_z) * M_SHARD + my_x * HALF + i * CH

        def x_recv_rows(i):
            return (1 - my_z) * M_SHARD + (1 - my_x) * HALF + i * CH

        z_rdmas = []
        for i in range(N_CHUNKS):
            r = pltpu.make_async_remote_copy(
                src_ref=x_ref.at[pl.ds(my_x * HALF + i * CH, CH), :],
                dst_ref=out_ref.at[pl.ds(my_half_rows(i), CH), :],
                send_sem=z_send_sems.at[i],
                recv_sem=z_recv_sems.at[i],
                device_id=z_peer,
                device_id_type=pl.DeviceIdType.MESH,
            )
            r.start()
            z_rdmas.append(r)

        x_rdmas = []
        for j in range(N_CHUNKS):
            z_rdmas[j].wait_recv()
            r = pltpu.make_async_remote_copy(
                src_ref=out_ref.at[pl.ds(z_recv_rows(j), CH), :],
                dst_ref=out_ref.at[pl.ds(z_recv_rows(j), CH), :],
                send_sem=x_send_sems.at[j],
                recv_sem=x_recv_sems.at[j],
                device_id=x_peer,
                device_id_type=pl.DeviceIdType.MESH,
            )
            r.start()
            x_rdmas.append(r)

        for j in range(N_CHUNKS):
            recv = pltpu.make_async_remote_copy(
                src_ref=out_ref.at[pl.ds(x_recv_rows(j), CH), :],
                dst_ref=out_ref.at[pl.ds(x_recv_rows(j), CH), :],
                send_sem=x_send_sems.at[j],
                recv_sem=x_recv_sems.at[j],
                device_id=x_peer,
                device_id_type=pl.DeviceIdType.MESH,
            )
            recv.wait_recv()
        for j in range(N_CHUNKS):
            z_rdmas[j].wait_send()
            x_rdmas[j].wait_send()
        local.wait()

    return pl.pallas_call(
        body,
        out_shape=jax.ShapeDtypeStruct((N_Z * M_SHARD, N), jnp.bfloat16),
        in_specs=[pl.BlockSpec(memory_space=pl.ANY)],
        out_specs=pl.BlockSpec(memory_space=pl.ANY),
        scratch_shapes=[
            pltpu.SemaphoreType.DMA,
            pltpu.SemaphoreType.DMA((N_CHUNKS,)),
            pltpu.SemaphoreType.DMA((N_CHUNKS,)),
            pltpu.SemaphoreType.DMA((N_CHUNKS,)),
            pltpu.SemaphoreType.DMA((N_CHUNKS,)),
        ],
        compiler_params=pltpu.CompilerParams(collective_id=0),
    )(x_bf)
